# baseline (speedup 1.0000x reference)
import functools

import jax
import jax.numpy as jnp
from jax import lax
from jax.experimental import pallas as pl
from jax.experimental.pallas import tpu as pltpu
from jax.experimental.pallas import tpu_sc as plsc

N = 10000
E = 320000
D = 128
NC = 2
NS = 16
NW = NC * NS
EPW = E // NW
CHUNK = 128
NCH = 80
HCH = NCH // 2
PADN = NCH * CHUNK - EPW
NSP = 10016
GARBAGE = 10008
CWID = 128
ZROWS = 640
LAST_Z = NSP - (NS - 1) * ZROWS
LAST_C = N - (NS - 1) * ZROWS

_mesh = plsc.VectorSubcoreMesh(core_axis_name="c", subcore_axis_name="s")


def _zero_stripe(z_hbm, zbuf, sp, s):
    pltpu.sync_copy(z_hbm, zbuf)
    z0 = s * ZROWS

    @pl.when(s < NS - 1)
    def _():
        for k in range(ZROWS // CHUNK):
            pltpu.sync_copy(zbuf, sp.at[pl.ds(z0 + k * CHUNK, CHUNK)])

    @pl.when(s == NS - 1)
    def _():
        for k in range(LAST_Z // CHUNK):
            pltpu.sync_copy(zbuf,
                            sp.at[pl.ds((NS - 1) * ZROWS + k * CHUNK, CHUNK)])
        rem = LAST_Z % CHUNK
        if rem:
            pltpu.sync_copy(
                zbuf.at[pl.ds(0, rem)],
                sp.at[pl.ds((NS - 1) * ZROWS + (LAST_Z // CHUNK) * CHUNK,
                            rem)])


def _copyout_stripe(sp, out, c, s):
    @pl.when(s < NS - 1)
    def _():
        pltpu.sync_copy(sp.at[pl.ds(s * ZROWS, ZROWS)],
                        out.at[c, pl.ds(s * ZROWS, ZROWS)])

    @pl.when(s == NS - 1)
    def _():
        pltpu.sync_copy(sp.at[pl.ds((NS - 1) * ZROWS, LAST_C)],
                        out.at[c, pl.ds((NS - 1) * ZROWS, LAST_C)])


def _sc_agg_body(h_hbm, z_hbm, sp_hbm, dp_hbm,
                 p_out, p_sp, s_v, d_v, rows_v, sem_a, sem_b,
                 sem_s0, sem_s1):
    c = lax.axis_index("c")
    s = lax.axis_index("s")
    w = c * NS + s

    _zero_stripe(z_hbm, rows_v.at[0], p_sp, s)
    pltpu.sync_copy(sp_hbm.at[w, pl.ds(0, HCH)], s_v)
    pltpu.sync_copy(dp_hbm.at[w, pl.ds(0, HCH)], d_v)
    plsc.subcore_barrier()

    for half in range(2):
        if half == 1:
            pltpu.sync_copy(sp_hbm.at[w, pl.ds(HCH, HCH)], s_v)
            pltpu.sync_copy(dp_hbm.at[w, pl.ds(HCH, HCH)], d_v)
        pltpu.async_copy(h_hbm.at[s_v.at[0]], rows_v.at[0], sem_a)
        pltpu.async_copy(h_hbm.at[s_v.at[1]], rows_v.at[1], sem_b)

        @pl.loop(0, HCH // 2)
        def _(p):
            c0 = 2 * p
            c1 = c0 + 1
            pltpu.make_async_copy(h_hbm.at[s_v.at[c0]], rows_v.at[0],
                                  sem_a).wait()
            pltpu.async_copy(rows_v.at[0], p_sp.at[d_v.at[c0]], sem_s0,
                             add=True)
            pltpu.make_async_copy(h_hbm.at[s_v.at[c1]], rows_v.at[1],
                                  sem_b).wait()
            pltpu.async_copy(rows_v.at[1], p_sp.at[d_v.at[c1]], sem_s1,
                             add=True)
            pltpu.make_async_copy(rows_v.at[0], p_sp.at[d_v.at[c0]],
                                  sem_s0).wait()

            @pl.when(p < HCH // 2 - 1)
            def _():
                pltpu.async_copy(h_hbm.at[s_v.at[c0 + 2]], rows_v.at[0],
                                 sem_a)

            pltpu.make_async_copy(rows_v.at[1], p_sp.at[d_v.at[c1]],
                                  sem_s1).wait()

            @pl.when(p < HCH // 2 - 1)
            def _():
                pltpu.async_copy(h_hbm.at[s_v.at[c1 + 2]], rows_v.at[1],
                                 sem_b)

    plsc.subcore_barrier()
    _copyout_stripe(p_sp, p_out, c, s)


_sc_agg = pl.kernel(
    _sc_agg_body,
    out_type=jax.ShapeDtypeStruct((NC, N, D), jnp.float32),
    mesh=_mesh,
    scratch_types=[
        pltpu.VMEM_SHARED((NSP, D), jnp.float32),
        pltpu.VMEM((HCH, CHUNK), jnp.int32),
        pltpu.VMEM((HCH, CHUNK), jnp.int32),
        pltpu.VMEM((2, CHUNK, D), jnp.float32),
        pltpu.SemaphoreType.DMA,
        pltpu.SemaphoreType.DMA,
        pltpu.SemaphoreType.DMA,
        pltpu.SemaphoreType.DMA,
    ],
    name="sage_sc_agg",
)


def _sc_cnt_body(dp_hbm, z_hbm, ones_hbm,
                 cw_out, cw_sp, d_v, ones_v, zbuf, sem_s):
    c = lax.axis_index("c")
    s = lax.axis_index("s")
    w = c * NS + s

    _zero_stripe(z_hbm, zbuf, cw_sp, s)
    pltpu.sync_copy(dp_hbm.at[w], d_v)
    pltpu.sync_copy(ones_hbm, ones_v)
    plsc.subcore_barrier()

    @pl.loop(0, NCH // 8)
    def _(q):
        base = 8 * q
        for j in range(8):
            pltpu.async_copy(ones_v, cw_sp.at[d_v.at[base + j]], sem_s,
                             add=True)
        for _j in range(8):
            pltpu.make_async_copy(ones_v, cw_sp.at[d_v.at[base]],
                                  sem_s).wait()

    plsc.subcore_barrier()
    _copyout_stripe(cw_sp, cw_out, c, s)


_sc_cnt = pl.kernel(
    _sc_cnt_body,
    out_type=jax.ShapeDtypeStruct((NC, N, CWID), jnp.float32),
    mesh=_mesh,
    scratch_types=[
        pltpu.VMEM_SHARED((NSP, CWID), jnp.float32),
        pltpu.VMEM((NCH, CHUNK), jnp.int32),
        pltpu.VMEM((CHUNK, CWID), jnp.float32),
        pltpu.VMEM((CHUNK, D), jnp.float32),
        pltpu.SemaphoreType.DMA,
    ],
    name="sage_sc_counts",
)

BT = 1000


def _tc_body_first(p_ref, cw_ref, h_ref, wl_ref, wr_ref, bl_ref,
                   out_ref, inv_ref):
    m = p_ref[0] + p_ref[1]
    cnt = cw_ref[0, :, 0:1] + cw_ref[1, :, 0:1]
    invb = jnp.broadcast_to(1.0 / jnp.maximum(cnt, 1.0), (BT, D))
    agg = lax.dot_general(m, wl_ref[...], (((1,), (1,)), ((), ())),
                          preferred_element_type=jnp.float32)
    out = (agg * invb + bl_ref[...] +
           lax.dot_general(h_ref[...], wr_ref[...], (((1,), (1,)), ((), ())),
                           preferred_element_type=jnp.float32))
    out_ref[...] = jnp.maximum(out, 0.0)
    inv_ref[...] = invb


def _tc_body(relu, p_ref, inv_ref, h_ref, wl_ref, wr_ref, bl_ref, out_ref):
    m = p_ref[0] + p_ref[1]
    agg = lax.dot_general(m, wl_ref[...], (((1,), (1,)), ((), ())),
                          preferred_element_type=jnp.float32)
    out = (agg * inv_ref[...] + bl_ref[...] +
           lax.dot_general(h_ref[...], wr_ref[...], (((1,), (1,)), ((), ())),
                           preferred_element_type=jnp.float32))
    out_ref[...] = jnp.maximum(out, 0.0) if relu else out


_w_spec = pl.BlockSpec((D, D), lambda i: (0, 0))
_b_spec = pl.BlockSpec((1, D), lambda i: (0, 0))
_h_spec = pl.BlockSpec((BT, D), lambda i: (i, 0))
_p_spec = pl.BlockSpec((NC, BT, D), lambda i: (0, i, 0))

_tc_first = pl.pallas_call(
    _tc_body_first,
    grid=(N // BT,),
    in_specs=[_p_spec, _p_spec, _h_spec, _w_spec, _w_spec, _b_spec],
    out_specs=[_h_spec, _h_spec],
    out_shape=[jax.ShapeDtypeStruct((N, D), jnp.float32),
               jax.ShapeDtypeStruct((N, D), jnp.float32)],
)

_tc_mid = pl.pallas_call(
    functools.partial(_tc_body, True),
    grid=(N // BT,),
    in_specs=[_p_spec, _h_spec, _h_spec, _w_spec, _w_spec, _b_spec],
    out_specs=_h_spec,
    out_shape=jax.ShapeDtypeStruct((N, D), jnp.float32),
)

_tc_last = pl.pallas_call(
    functools.partial(_tc_body, False),
    grid=(N // BT,),
    in_specs=[_p_spec, _h_spec, _h_spec, _w_spec, _w_spec, _b_spec],
    out_specs=_h_spec,
    out_shape=jax.ShapeDtypeStruct((N, D), jnp.float32),
)


def kernel(x, edge_attr, edge_index, Wl0, bl0, Wr0, Wl1, bl1, Wr1,
           Wl2, bl2, Wr2):
    del edge_attr
    ei = edge_index.astype(jnp.int32)
    src = ei[0].reshape(NW, EPW)
    dst = ei[1].reshape(NW, EPW)
    pad_src = (jnp.arange(NW * PADN, dtype=jnp.int32) * 97 % N).reshape(NW, PADN)
    pad_dst = jnp.full((NW, PADN), GARBAGE, jnp.int32)
    sp = jnp.concatenate([src, pad_src], axis=1).reshape(NW, NCH, CHUNK)
    dp = jnp.concatenate([dst, pad_dst], axis=1).reshape(NW, NCH, CHUNK)
    z128 = jnp.zeros((CHUNK, D), jnp.float32)
    ones128 = jnp.ones((CHUNK, CWID), jnp.float32)

    cw = _sc_cnt(dp, z128, ones128)
    p0 = _sc_agg(x, z128, sp, dp)
    h1, invb = _tc_first(p0, cw, x, Wl0, Wr0, bl0.reshape(1, D))
    p1 = _sc_agg(h1, z128, sp, dp)
    h2 = _tc_mid(p1, invb, h1, Wl1, Wr1, bl1.reshape(1, D))
    p2 = _sc_agg(h2, z128, sp, dp)
    return _tc_last(p2, invb, h2, Wl2, Wr2, bl2.reshape(1, D))


# speedup vs baseline: 1.2582x; 1.2582x over previous
import functools

import jax
import jax.numpy as jnp
from jax import lax
from jax.experimental import pallas as pl
from jax.experimental.pallas import tpu as pltpu
from jax.experimental.pallas import tpu_sc as plsc

N = 10000
E = 320000
D = 128
NC = 2
NS = 16
NW = NC * NS
EPW = E // NW
CHUNK = 128
NCH = 80
HCH = NCH // 2
PADN = NCH * CHUNK - EPW
NSP = 10016
GARBAGE = 10008
CWID = 128
ZROWS = 640
LAST_Z = NSP - (NS - 1) * ZROWS
LAST_C = N - (NS - 1) * ZROWS

_mesh = plsc.VectorSubcoreMesh(core_axis_name="c", subcore_axis_name="s")


def _zero_stripe(z_hbm, sp, s):
    @pl.when(s < NS - 1)
    def _():
        pltpu.sync_copy(z_hbm.at[pl.ds(s * ZROWS, ZROWS)],
                        sp.at[pl.ds(s * ZROWS, ZROWS)])

    @pl.when(s == NS - 1)
    def _():
        pltpu.sync_copy(z_hbm.at[pl.ds((NS - 1) * ZROWS, LAST_Z)],
                        sp.at[pl.ds((NS - 1) * ZROWS, LAST_Z)])


def _copyout_stripe(sp, out, c, s):
    @pl.when(s < NS - 1)
    def _():
        pltpu.sync_copy(sp.at[pl.ds(s * ZROWS, ZROWS)],
                        out.at[c, pl.ds(s * ZROWS, ZROWS)])

    @pl.when(s == NS - 1)
    def _():
        pltpu.sync_copy(sp.at[pl.ds((NS - 1) * ZROWS, LAST_C)],
                        out.at[c, pl.ds((NS - 1) * ZROWS, LAST_C)])


def _sc_agg_body(h_hbm, z_hbm, sp_hbm, dp_hbm,
                 p_out, p_sp, s_v, d_v, rows_v, sem_a, sem_b, sem_z):
    c = lax.axis_index("c")
    s = lax.axis_index("s")
    w = c * NS + s
    z0 = s * ZROWS

    @pl.when(s < NS - 1)
    def _():
        pltpu.async_copy(z_hbm.at[pl.ds(z0, ZROWS)],
                         p_sp.at[pl.ds(z0, ZROWS)], sem_z)

    @pl.when(s == NS - 1)
    def _():
        pltpu.async_copy(z_hbm.at[pl.ds((NS - 1) * ZROWS, LAST_Z)],
                         p_sp.at[pl.ds((NS - 1) * ZROWS, LAST_Z)], sem_z)

    pltpu.sync_copy(sp_hbm.at[w, pl.ds(0, HCH)], s_v)
    pltpu.sync_copy(dp_hbm.at[w, pl.ds(0, HCH)], d_v)

    @pl.when(s < NS - 1)
    def _():
        pltpu.make_async_copy(z_hbm.at[pl.ds(z0, ZROWS)],
                              p_sp.at[pl.ds(z0, ZROWS)], sem_z).wait()

    @pl.when(s == NS - 1)
    def _():
        pltpu.make_async_copy(z_hbm.at[pl.ds((NS - 1) * ZROWS, LAST_Z)],
                              p_sp.at[pl.ds((NS - 1) * ZROWS, LAST_Z)],
                              sem_z).wait()

    plsc.subcore_barrier()

    for half in range(2):
        if half == 1:
            pltpu.sync_copy(sp_hbm.at[w, pl.ds(HCH, HCH)], s_v)
            pltpu.sync_copy(dp_hbm.at[w, pl.ds(HCH, HCH)], d_v)
        pltpu.async_copy(h_hbm.at[s_v.at[0]], rows_v.at[0], sem_a)
        pltpu.async_copy(h_hbm.at[s_v.at[1]], rows_v.at[1], sem_b)

        @pl.loop(0, HCH // 2)
        def _(p):
            c0 = 2 * p
            c1 = c0 + 1
            pltpu.make_async_copy(h_hbm.at[s_v.at[c0]], rows_v.at[0],
                                  sem_a).wait()
            pltpu.sync_copy(rows_v.at[0], p_sp.at[d_v.at[c0]], add=True)

            @pl.when(p < HCH // 2 - 1)
            def _():
                pltpu.async_copy(h_hbm.at[s_v.at[c0 + 2]], rows_v.at[0],
                                 sem_a)

            pltpu.make_async_copy(h_hbm.at[s_v.at[c1]], rows_v.at[1],
                                  sem_b).wait()
            pltpu.sync_copy(rows_v.at[1], p_sp.at[d_v.at[c1]], add=True)

            @pl.when(p < HCH // 2 - 1)
            def _():
                pltpu.async_copy(h_hbm.at[s_v.at[c1 + 2]], rows_v.at[1],
                                 sem_b)

    plsc.subcore_barrier()
    _copyout_stripe(p_sp, p_out, c, s)


_sc_agg = pl.kernel(
    _sc_agg_body,
    out_type=jax.ShapeDtypeStruct((NC, N, D), jnp.float32),
    mesh=_mesh,
    scratch_types=[
        pltpu.VMEM_SHARED((NSP, D), jnp.float32),
        pltpu.VMEM((HCH, CHUNK), jnp.int32),
        pltpu.VMEM((HCH, CHUNK), jnp.int32),
        pltpu.VMEM((2, CHUNK, D), jnp.float32),
        pltpu.SemaphoreType.DMA,
        pltpu.SemaphoreType.DMA,
        pltpu.SemaphoreType.DMA,
    ],
    name="sage_sc_agg",
)


def _sc_cnt_body(dp_hbm, z_hbm, ones_hbm,
                 cw_out, cw_sp, d_v, ones_v, sem_s):
    c = lax.axis_index("c")
    s = lax.axis_index("s")
    w = c * NS + s

    _zero_stripe(z_hbm, cw_sp, s)
    pltpu.sync_copy(dp_hbm.at[w], d_v)
    pltpu.sync_copy(ones_hbm, ones_v)
    plsc.subcore_barrier()

    @pl.loop(0, NCH // 8)
    def _(q):
        base = 8 * q
        for j in range(8):
            pltpu.async_copy(ones_v, cw_sp.at[d_v.at[base + j]], sem_s,
                             add=True)
        for _j in range(8):
            pltpu.make_async_copy(ones_v, cw_sp.at[d_v.at[base]],
                                  sem_s).wait()

    plsc.subcore_barrier()
    _copyout_stripe(cw_sp, cw_out, c, s)


_sc_cnt = pl.kernel(
    _sc_cnt_body,
    out_type=jax.ShapeDtypeStruct((NC, N, CWID), jnp.float32),
    mesh=_mesh,
    scratch_types=[
        pltpu.VMEM_SHARED((NSP, CWID), jnp.float32),
        pltpu.VMEM((NCH, CHUNK), jnp.int32),
        pltpu.VMEM((CHUNK, CWID), jnp.float32),
        pltpu.SemaphoreType.DMA,
    ],
    name="sage_sc_counts",
)

BT = 5000


def _tc_root_body(h_ref, wr_ref, bl_ref, r_ref):
    r_ref[...] = bl_ref[...] + lax.dot_general(
        h_ref[...], wr_ref[...], (((1,), (1,)), ((), ())),
        preferred_element_type=jnp.float32)


def _tc_body_first(p_ref, cw_ref, r_ref, wl_ref, out_ref, inv_ref):
    m = p_ref[0] + p_ref[1]
    cnt = cw_ref[0, :, 0:1] + cw_ref[1, :, 0:1]
    invb = jnp.broadcast_to(1.0 / jnp.maximum(cnt, 1.0), (BT, D))
    agg = lax.dot_general(m, wl_ref[...], (((1,), (1,)), ((), ())),
                          preferred_element_type=jnp.float32)
    out_ref[...] = jnp.maximum(agg * invb + r_ref[...], 0.0)
    inv_ref[...] = invb


def _tc_body(relu, p_ref, inv_ref, r_ref, wl_ref, out_ref):
    m = p_ref[0] + p_ref[1]
    agg = lax.dot_general(m, wl_ref[...], (((1,), (1,)), ((), ())),
                          preferred_element_type=jnp.float32)
    out = agg * inv_ref[...] + r_ref[...]
    out_ref[...] = jnp.maximum(out, 0.0) if relu else out


_w_spec = pl.BlockSpec((D, D), lambda i: (0, 0))
_b_spec = pl.BlockSpec((1, D), lambda i: (0, 0))
_h_spec = pl.BlockSpec((BT, D), lambda i: (i, 0))
_p_spec = pl.BlockSpec((NC, BT, D), lambda i: (0, i, 0))

_tc_root = pl.pallas_call(
    _tc_root_body,
    grid=(N // BT,),
    in_specs=[_h_spec, _w_spec, _b_spec],
    out_specs=_h_spec,
    out_shape=jax.ShapeDtypeStruct((N, D), jnp.float32),
)

_tc_first = pl.pallas_call(
    _tc_body_first,
    grid=(N // BT,),
    in_specs=[_p_spec, _p_spec, _h_spec, _w_spec],
    out_specs=[_h_spec, _h_spec],
    out_shape=[jax.ShapeDtypeStruct((N, D), jnp.float32),
               jax.ShapeDtypeStruct((N, D), jnp.float32)],
)

_tc_mid = pl.pallas_call(
    functools.partial(_tc_body, True),
    grid=(N // BT,),
    in_specs=[_p_spec, _h_spec, _h_spec, _w_spec],
    out_specs=_h_spec,
    out_shape=jax.ShapeDtypeStruct((N, D), jnp.float32),
)

_tc_last = pl.pallas_call(
    functools.partial(_tc_body, False),
    grid=(N // BT,),
    in_specs=[_p_spec, _h_spec, _h_spec, _w_spec],
    out_specs=_h_spec,
    out_shape=jax.ShapeDtypeStruct((N, D), jnp.float32),
)


def kernel(x, edge_attr, edge_index, Wl0, bl0, Wr0, Wl1, bl1, Wr1,
           Wl2, bl2, Wr2):
    del edge_attr
    ei = edge_index.astype(jnp.int32)
    src = ei[0].reshape(NW, EPW)
    dst = ei[1].reshape(NW, EPW)
    pad_src = (jnp.arange(NW * PADN, dtype=jnp.int32) * 97 % N).reshape(NW, PADN)
    pad_dst = jnp.full((NW, PADN), GARBAGE, jnp.int32)
    sp = jnp.concatenate([src, pad_src], axis=1).reshape(NW, NCH, CHUNK)
    dp = jnp.concatenate([dst, pad_dst], axis=1).reshape(NW, NCH, CHUNK)
    z128 = jnp.zeros((NSP, D), jnp.float32)
    ones128 = jnp.ones((CHUNK, CWID), jnp.float32)

    cw = _sc_cnt(dp, z128, ones128)
    p0 = _sc_agg(x, z128, sp, dp)
    r0 = _tc_root(x, Wr0, bl0.reshape(1, D))
    h1, invb = _tc_first(p0, cw, r0, Wl0)
    p1 = _sc_agg(h1, z128, sp, dp)
    r1 = _tc_root(h1, Wr1, bl1.reshape(1, D))
    h2 = _tc_mid(p1, invb, r1, Wl1)
    p2 = _sc_agg(h2, z128, sp, dp)
    r2 = _tc_root(h2, Wr2, bl2.reshape(1, D))
    return _tc_last(p2, invb, r2, Wl2)
